# Initial kernel scaffold; baseline (speedup 1.0000x reference)
import jax
import jax.numpy as jnp
from jax.experimental import pallas as pl


def kernel(node_states, edges, kernel, kernel_attention):
    raise NotImplementedError("write your pallas kernel here")


# speedup vs baseline: 9.8503x; 9.8503x over previous
import functools

import jax
import jax.numpy as jnp
from jax import lax
from jax.experimental import pallas as pl
from jax.experimental.pallas import tpu as pltpu
from jax.experimental.pallas import tpu_sc as plsc

N_NODES = 10000
D = 128
E = 320000

NC = 2
NS = 16
NW = NC * NS
CH = 128
EDGES_PER_TILE = 10240
E_PAD = NW * EDGES_PER_TILE
N_CHUNKS = EDGES_PER_TILE // CH
N_SLABS = 79
N_PAD = N_SLABS * CH

ROWS_BLK = 1000


def _tc_transform_body(x_ref, w_ref, a_ref, h_ref, s_ref):
    h = jnp.dot(x_ref[...], w_ref[...], preferred_element_type=jnp.float32)
    h_ref[...] = h
    s_ref[...] = jnp.dot(h, a_ref[...], preferred_element_type=jnp.float32)


def _tc_transform(x, w, a_mat):
    n_blocks = N_NODES // ROWS_BLK
    return pl.pallas_call(
        _tc_transform_body,
        grid=(n_blocks,),
        in_specs=[
            pl.BlockSpec((ROWS_BLK, D), lambda i: (i, 0)),
            pl.BlockSpec((D, D), lambda i: (0, 0)),
            pl.BlockSpec((D, D), lambda i: (0, 0)),
        ],
        out_specs=[
            pl.BlockSpec((ROWS_BLK, D), lambda i: (i, 0)),
            pl.BlockSpec((ROWS_BLK, D), lambda i: (i, 0)),
        ],
        out_shape=[
            jax.ShapeDtypeStruct((N_NODES, D), jnp.float32),
            jax.ShapeDtypeStruct((N_NODES, D), jnp.float32),
        ],
    )(x, w, a_mat)


def _sc_body(src_hbm, dst_hbm, h_hbm, s1_hbm, s2_hbm,
             outp_hbm, denp_hbm,
             s1_v, s2_v, srcv, dstv, rows, wbuf, zrow, out_sh, den_sh, sem):
    cid = lax.axis_index("c")
    sid = lax.axis_index("s")
    wid = sid * NC + cid

    pltpu.sync_copy(s1_hbm, s1_v)
    pltpu.sync_copy(s2_hbm, s2_v)

    zero16 = jnp.zeros((16,), jnp.float32)

    def _zero_rows(r, _):
        for k in range(D // 16):
            rows[r, pl.ds(k * 16, 16)] = zero16
        return 0

    lax.fori_loop(0, CH, _zero_rows, 0)
    for j in range(CH // 16):
        zrow[pl.ds(j * 16, 16)] = zero16
        wbuf[pl.ds(j * 16, 16)] = zero16

    for k in range((N_SLABS + NS - 1) // NS):
        s = sid + NS * k

        @pl.when(s < N_SLABS)
        def _():
            pltpu.sync_copy(rows, out_sh.at[pl.ds(s * CH, CH)])
            pltpu.sync_copy(zrow, den_sh.at[pl.ds(s * CH, CH)])

    plsc.subcore_barrier()

    ebase0 = wid * EDGES_PER_TILE

    def _chunk(c, _):
        base = ebase0 + c * CH
        pltpu.sync_copy(src_hbm.at[pl.ds(base, CH)], srcv)
        pltpu.sync_copy(dst_hbm.at[pl.ds(base, CH)], dstv)
        gather = pltpu.async_copy(h_hbm.at[dstv], rows, sem)

        def _wcalc(j, _):
            sidx = srcv[pl.ds(j * 16, 16)]
            didx = dstv[pl.ds(j * 16, 16)]
            z = (plsc.load_gather(s1_v, [sidx])
                 + plsc.load_gather(s2_v, [didx]))
            z = jnp.where(z >= 0.0, z, 0.2 * z)
            z = jnp.clip(z, -2.0, 2.0)
            w = jnp.exp(z)
            eid = base + j * 16 + lax.iota(jnp.int32, 16)
            w = jnp.where(eid < E, w, 0.0)
            wbuf[pl.ds(j * 16, 16)] = w
            return 0

        lax.fori_loop(0, CH // 16, _wcalc, 0)
        gather.wait()

        def _scale(r, _):
            wv = plsc.load_gather(wbuf, [jnp.full((16,), r, jnp.int32)])
            for k in range(D // 16):
                sl = pl.ds(k * 16, 16)
                rows[r, sl] = rows[r, sl] * wv
            return 0

        lax.fori_loop(0, CH, _scale, 0)

        pltpu.sync_copy(wbuf, den_sh.at[srcv], add=True)
        pltpu.sync_copy(rows, out_sh.at[srcv], add=True)
        return 0

    lax.fori_loop(0, N_CHUNKS, _chunk, 0)

    plsc.subcore_barrier()

    for k in range((N_SLABS + NS - 1) // NS):
        s = sid + NS * k

        @pl.when(s < N_SLABS)
        def _():
            pltpu.sync_copy(out_sh.at[pl.ds(s * CH, CH)],
                            outp_hbm.at[cid, pl.ds(s * CH, CH)])
            pltpu.sync_copy(den_sh.at[pl.ds(s * CH, CH)],
                            denp_hbm.at[cid, pl.ds(s * CH, CH)])


def _sc_aggregate(src, dst, h, s1, s2):
    mesh = plsc.VectorSubcoreMesh(core_axis_name="c", subcore_axis_name="s")
    fn = pl.kernel(
        _sc_body,
        out_type=[
            jax.ShapeDtypeStruct((NC, N_PAD, D), jnp.float32),
            jax.ShapeDtypeStruct((NC, N_PAD), jnp.float32),
        ],
        mesh=mesh,
        scratch_types=[
            pltpu.VMEM((N_NODES,), jnp.float32),
            pltpu.VMEM((N_NODES,), jnp.float32),
            pltpu.VMEM((CH,), jnp.int32),
            pltpu.VMEM((CH,), jnp.int32),
            pltpu.VMEM((CH, D), jnp.float32),
            pltpu.VMEM((CH,), jnp.float32),
            pltpu.VMEM((CH,), jnp.float32),
            pltpu.VMEM_SHARED((N_PAD, D), jnp.float32),
            pltpu.VMEM_SHARED((N_PAD,), jnp.float32),
            pltpu.SemaphoreType.DMA,
        ],
        compiler_params=pltpu.CompilerParams(needs_layout_passes=False),
    )
    return fn(src, dst, h, s1, s2)


def _tc_combine_body(o0_ref, o1_ref, d0_ref, d1_ref, out_ref):
    den = d0_ref[...] + d1_ref[...]
    num = o0_ref[...] + o1_ref[...]
    out_ref[...] = jnp.where(den > 0.0, num / jnp.where(den > 0.0, den, 1.0),
                             0.0)


def _tc_combine(o0, o1, d0, d1):
    n_blocks = N_NODES // ROWS_BLK
    return pl.pallas_call(
        _tc_combine_body,
        grid=(n_blocks,),
        in_specs=[
            pl.BlockSpec((ROWS_BLK, D), lambda i: (i, 0)),
            pl.BlockSpec((ROWS_BLK, D), lambda i: (i, 0)),
            pl.BlockSpec((ROWS_BLK, 1), lambda i: (i, 0)),
            pl.BlockSpec((ROWS_BLK, 1), lambda i: (i, 0)),
        ],
        out_specs=pl.BlockSpec((ROWS_BLK, D), lambda i: (i, 0)),
        out_shape=jax.ShapeDtypeStruct((N_NODES, D), jnp.float32),
    )(o0, o1, d0, d1)


def kernel(node_states, edges, kernel, kernel_attention):
    w = kernel.astype(jnp.float32)
    a = kernel_attention.astype(jnp.float32)
    a_mat = jnp.pad(jnp.concatenate([a[:D], a[D:]], axis=1),
                    ((0, 0), (0, D - 2)))

    edges_i = edges.astype(jnp.int32)
    pad = E_PAD - E
    src = jnp.concatenate([edges_i[:, 0], jnp.zeros((pad,), jnp.int32)])
    dst = jnp.concatenate([edges_i[:, 1], jnp.zeros((pad,), jnp.int32)])

    h, s_cat = _tc_transform(node_states.astype(jnp.float32), w, a_mat)
    s1 = s_cat[:, 0]
    s2 = s_cat[:, 1]

    outp, denp = _sc_aggregate(src, dst, h, s1, s2)

    return _tc_combine(outp[0, :N_NODES], outp[1, :N_NODES],
                       denp[0, :N_NODES, None], denp[1, :N_NODES, None])


# speedup vs baseline: 13.5979x; 1.3805x over previous
import functools

import jax
import jax.numpy as jnp
from jax import lax
from jax.experimental import pallas as pl
from jax.experimental.pallas import tpu as pltpu
from jax.experimental.pallas import tpu_sc as plsc

N_NODES = 10000
D = 128
E = 320000

NC = 2
NS = 16
NW = NC * NS
CH = 128
EDGES_PER_TILE = 10240
E_PAD = NW * EDGES_PER_TILE
N_CHUNKS = EDGES_PER_TILE // CH
N_SLABS = 79
N_PAD = N_SLABS * CH

ROWS_BLK = 1000


def _tc_transform_body(x_ref, w_ref, a_ref, h_ref, s_ref):
    h = jnp.dot(x_ref[...], w_ref[...], preferred_element_type=jnp.float32)
    h_ref[...] = h
    s_ref[...] = jnp.dot(h, a_ref[...], preferred_element_type=jnp.float32)


def _tc_transform(x, w, a_mat):
    n_blocks = N_NODES // ROWS_BLK
    return pl.pallas_call(
        _tc_transform_body,
        grid=(n_blocks,),
        in_specs=[
            pl.BlockSpec((ROWS_BLK, D), lambda i: (i, 0)),
            pl.BlockSpec((D, D), lambda i: (0, 0)),
            pl.BlockSpec((D, D), lambda i: (0, 0)),
        ],
        out_specs=[
            pl.BlockSpec((ROWS_BLK, D), lambda i: (i, 0)),
            pl.BlockSpec((ROWS_BLK, D), lambda i: (i, 0)),
        ],
        out_shape=[
            jax.ShapeDtypeStruct((N_NODES, D), jnp.float32),
            jax.ShapeDtypeStruct((N_NODES, D), jnp.float32),
        ],
    )(x, w, a_mat)


def _sc_body(src_hbm, dst_hbm, h_hbm, s1_hbm, s2_hbm,
             outp_hbm, denp_hbm,
             srcv, dstv, rows, s1g, s2g, wbuf, zrow, out_sh, den_sh,
             gsem0, gsem1):
    gsem = (gsem0, gsem1)
    cid = lax.axis_index("c")
    sid = lax.axis_index("s")
    wid = sid * NC + cid

    zero16 = jnp.zeros((16,), jnp.float32)

    def _zero_rows(r, _):
        for k in range(D // 16):
            rows[0, r, pl.ds(k * 16, 16)] = zero16
        return 0

    lax.fori_loop(0, CH, _zero_rows, 0)
    for j in range(CH // 16):
        zrow[pl.ds(j * 16, 16)] = zero16

    for k in range((N_SLABS + NS - 1) // NS):
        s = sid + NS * k

        @pl.when(s < N_SLABS)
        def _():
            pltpu.sync_copy(rows.at[0], out_sh.at[pl.ds(s * CH, CH)])
            pltpu.sync_copy(zrow, den_sh.at[pl.ds(s * CH, CH)])

    plsc.subcore_barrier()

    ebase0 = wid * EDGES_PER_TILE

    def _stage(c, b):
        base = ebase0 + c * CH
        pltpu.sync_copy(src_hbm.at[pl.ds(base, CH)], srcv.at[b])
        pltpu.sync_copy(dst_hbm.at[pl.ds(base, CH)], dstv.at[b])
        pltpu.async_copy(s1_hbm.at[srcv.at[b]], s1g.at[b], gsem[b])
        pltpu.async_copy(s2_hbm.at[dstv.at[b]], s2g.at[b], gsem[b])
        pltpu.async_copy(h_hbm.at[dstv.at[b]], rows.at[b], gsem[b])

    def _drain(b):
        pltpu.make_async_copy(s1_hbm.at[srcv.at[b]], s1g.at[b],
                              gsem[b]).wait()
        pltpu.make_async_copy(s2_hbm.at[dstv.at[b]], s2g.at[b],
                              gsem[b]).wait()
        pltpu.make_async_copy(h_hbm.at[dstv.at[b]], rows.at[b],
                              gsem[b]).wait()

    def _process(c, b):
        base = ebase0 + c * CH
        _drain(b)

        def _wcalc(j, _):
            sl = pl.ds(j * 16, 16)
            z = s1g[b, sl] + s2g[b, sl]
            z = jnp.where(z >= 0.0, z, 0.2 * z)
            z = jnp.clip(z, -2.0, 2.0)
            w = jnp.exp(z)
            eid = base + j * 16 + lax.iota(jnp.int32, 16)
            w = jnp.where(eid < E, w, 0.0)
            wbuf[b, sl] = w
            return 0

        lax.fori_loop(0, CH // 16, _wcalc, 0)

        def _scale(i, _):
            for dr in range(4):
                r = i * 4 + dr
                wv = plsc.load_gather(wbuf.at[b],
                                      [jnp.full((16,), r, jnp.int32)])
                for k in range(D // 16):
                    sl = pl.ds(k * 16, 16)
                    rows[b, r, sl] = rows[b, r, sl] * wv
            return 0

        lax.fori_loop(0, CH // 4, _scale, 0)

        pltpu.sync_copy(wbuf.at[b], den_sh.at[srcv.at[b]], add=True)
        pltpu.sync_copy(rows.at[b], out_sh.at[srcv.at[b]], add=True)

    _stage(0, 0)

    def _outer(g, _):
        for b in range(2):
            c = 2 * g + b
            _stage(c + 1, 1 - b)
            _process(c, b)
        return 0

    lax.fori_loop(0, N_CHUNKS // 2, _outer, 0)

    _drain(0)

    plsc.subcore_barrier()

    for k in range((N_SLABS + NS - 1) // NS):
        s = sid + NS * k

        @pl.when(s < N_SLABS)
        def _():
            pltpu.sync_copy(out_sh.at[pl.ds(s * CH, CH)],
                            outp_hbm.at[cid, pl.ds(s * CH, CH)])
            pltpu.sync_copy(den_sh.at[pl.ds(s * CH, CH)],
                            denp_hbm.at[cid, pl.ds(s * CH, CH)])


def _sc_aggregate(src, dst, h, s1, s2):
    mesh = plsc.VectorSubcoreMesh(core_axis_name="c", subcore_axis_name="s")
    fn = pl.kernel(
        _sc_body,
        out_type=[
            jax.ShapeDtypeStruct((NC, N_PAD, D), jnp.float32),
            jax.ShapeDtypeStruct((NC, N_PAD), jnp.float32),
        ],
        mesh=mesh,
        scratch_types=[
            pltpu.VMEM((2, CH), jnp.int32),
            pltpu.VMEM((2, CH), jnp.int32),
            pltpu.VMEM((2, CH, D), jnp.float32),
            pltpu.VMEM((2, CH), jnp.float32),
            pltpu.VMEM((2, CH), jnp.float32),
            pltpu.VMEM((2, CH), jnp.float32),
            pltpu.VMEM((CH,), jnp.float32),
            pltpu.VMEM_SHARED((N_PAD, D), jnp.float32),
            pltpu.VMEM_SHARED((N_PAD,), jnp.float32),
            pltpu.SemaphoreType.DMA,
            pltpu.SemaphoreType.DMA,
        ],
        compiler_params=pltpu.CompilerParams(needs_layout_passes=False),
    )
    return fn(src, dst, h, s1, s2)


def _tc_combine_body(o0_ref, o1_ref, d0_ref, d1_ref, out_ref):
    den = d0_ref[...] + d1_ref[...]
    num = o0_ref[...] + o1_ref[...]
    out_ref[...] = jnp.where(den > 0.0, num / jnp.where(den > 0.0, den, 1.0),
                             0.0)


def _tc_combine(o0, o1, d0, d1):
    n_blocks = N_NODES // ROWS_BLK
    return pl.pallas_call(
        _tc_combine_body,
        grid=(n_blocks,),
        in_specs=[
            pl.BlockSpec((ROWS_BLK, D), lambda i: (i, 0)),
            pl.BlockSpec((ROWS_BLK, D), lambda i: (i, 0)),
            pl.BlockSpec((ROWS_BLK, 1), lambda i: (i, 0)),
            pl.BlockSpec((ROWS_BLK, 1), lambda i: (i, 0)),
        ],
        out_specs=pl.BlockSpec((ROWS_BLK, D), lambda i: (i, 0)),
        out_shape=jax.ShapeDtypeStruct((N_NODES, D), jnp.float32),
    )(o0, o1, d0, d1)


def kernel(node_states, edges, kernel, kernel_attention):
    w = kernel.astype(jnp.float32)
    a = kernel_attention.astype(jnp.float32)
    a_mat = jnp.pad(jnp.concatenate([a[:D], a[D:]], axis=1),
                    ((0, 0), (0, D - 2)))

    edges_i = edges.astype(jnp.int32)
    pad = E_PAD - E + CH
    src = jnp.concatenate([edges_i[:, 0], jnp.zeros((pad,), jnp.int32)])
    dst = jnp.concatenate([edges_i[:, 1], jnp.zeros((pad,), jnp.int32)])

    h, s_cat = _tc_transform(node_states.astype(jnp.float32), w, a_mat)
    s1 = s_cat[:, 0]
    s2 = s_cat[:, 1]

    outp, denp = _sc_aggregate(src, dst, h, s1, s2)

    return _tc_combine(outp[0, :N_NODES], outp[1, :N_NODES],
                       denp[0, :N_NODES, None], denp[1, :N_NODES, None])
